# baseline (speedup 1.0000x reference)
import functools

import jax
import jax.numpy as jnp
from jax import lax
from jax.experimental import pallas as pl
from jax.experimental.pallas import tpu as pltpu
from jax.experimental.pallas import tpu_sc as plsc

NC, NS = 2, 16
V = 1000000
D = 64
B1, B2 = 4096, 200
B = B1 * B2
CPS = D // NC
PPT = B // NS
CB = 2048
NCH = PPT // CB

_MESH = plsc.VectorSubcoreMesh(
    core_axis_name="c", subcore_axis_name="s", num_cores=NC, num_subcores=NS
)


@functools.partial(
    pl.kernel,
    out_type=jax.ShapeDtypeStruct((B2 * D, B1), jnp.float32),
    mesh=_MESH,
    compiler_params=pltpu.CompilerParams(use_tc_tiling_on_sc=False),
    scratch_types=[
        pltpu.VMEM((PPT,), jnp.int32),
        pltpu.VMEM((CB,), jnp.float32),
        pltpu.VMEM((CB,), jnp.float32),
        pltpu.VMEM_SHARED((V,), jnp.float32),
        pltpu.SemaphoreType.DMA,
        pltpu.SemaphoreType.DMA,
        pltpu.SemaphoreType.DMA,
        pltpu.SemaphoreType.DMA,
        pltpu.SemaphoreType.DMA,
    ],
)
def _colgather(xt_hbm, tt_hbm, out_hbm, idx_v, gb0, gb1, colA,
               sg0, sg1, sw0, sw1, scol):
    cid = lax.axis_index("c")
    sid = lax.axis_index("s")
    p0 = pl.multiple_of(sid * PPT, PPT)
    pltpu.sync_copy(xt_hbm.at[pl.ds(p0, PPT)], idx_v)

    gbufs = (gb0, gb1)
    gsems = (sg0, sg1)
    wsems = (sw0, sw1)
    jbase = cid * CPS

    def load_col(jj):
        pltpu.async_copy(tt_hbm.at[jbase + jj], colA, scol)

    def wait_col(jj):
        pltpu.make_async_copy(tt_hbm.at[jbase + jj], colA, scol).wait()

    @pl.when(sid == 0)
    def _():
        load_col(0)
        wait_col(0)

    @pl.loop(0, CPS)
    def _cols(jj):
        plsc.subcore_barrier()
        j = jbase + jj

        def g_start(k, b):
            pltpu.async_copy(
                colA.at[idx_v.at[pl.ds(pl.multiple_of(k * CB, CB), CB)]],
                gbufs[b], gsems[b],
            )

        def g_wait(k, b):
            pltpu.make_async_copy(
                colA.at[idx_v.at[pl.ds(pl.multiple_of(k * CB, CB), CB)]],
                gbufs[b], gsems[b],
            ).wait()

        def w_dst(k):
            p = p0 + k * CB
            r = (p >> 12) * D + j
            b1 = pl.multiple_of(p & (B1 - 1), CB)
            return out_hbm.at[r, pl.ds(b1, CB)]

        def w_start(k, b):
            pltpu.async_copy(gbufs[b], w_dst(k), wsems[b])

        def w_wait(k, b):
            pltpu.make_async_copy(gbufs[b], w_dst(k), wsems[b]).wait()

        g_start(0, 0)
        g_start(1, 1)

        @pl.loop(0, NCH - 1, step=2)
        def _chunks(k):
            g_wait(k, 0)
            w_start(k, 0)
            g_wait(k + 1, 1)
            w_start(k + 1, 1)
            w_wait(k, 0)
            g_start(k + 2, 0)

            @pl.when(k + 3 < NCH)
            def _():
                w_wait(k + 1, 1)
                g_start(k + 3, 1)

        g_wait(NCH - 1, 0)
        plsc.subcore_barrier()

        @pl.when(sid == 0)
        def _():
            @pl.when(jj + 1 < CPS)
            def _():
                load_col(jj + 1)

        w_start(NCH - 1, 0)
        w_wait(NCH - 2, 1)
        w_wait(NCH - 1, 0)

        @pl.when(sid == 0)
        def _():
            @pl.when(jj + 1 < CPS)
            def _():
                wait_col(jj + 1)


def kernel(x, table):
    xt = x.T.reshape(-1)
    tt = table.T
    out2 = _colgather(xt, tt)
    out3 = out2.reshape(B2, D, B1)
    return jnp.transpose(out3, (2, 0, 1))


# speedup vs baseline: 7.6101x; 7.6101x over previous
import functools

import jax
import jax.numpy as jnp
from jax import lax
from jax.experimental import pallas as pl
from jax.experimental.pallas import tpu as pltpu
from jax.experimental.pallas import tpu_sc as plsc

NC, NS = 2, 16
V = 1000000
D = 64
B1, B2 = 4096, 200
B = B1 * B2
CPS = D // NC
PPT = B // NS
CB = 2048
NCH = PPT // CB

_MESH = plsc.VectorSubcoreMesh(
    core_axis_name="c", subcore_axis_name="s", num_cores=NC, num_subcores=NS
)


@functools.partial(
    pl.kernel,
    out_type=jax.ShapeDtypeStruct((B2 * D, B1), jnp.float32),
    mesh=_MESH,
    compiler_params=pltpu.CompilerParams(use_tc_tiling_on_sc=True),
    scratch_types=[
        pltpu.VMEM((PPT,), jnp.int32),
        pltpu.VMEM((CB,), jnp.float32),
        pltpu.VMEM((CB,), jnp.float32),
        pltpu.VMEM_SHARED((V,), jnp.float32),
        pltpu.SemaphoreType.DMA,
        pltpu.SemaphoreType.DMA,
        pltpu.SemaphoreType.DMA,
        pltpu.SemaphoreType.DMA,
        pltpu.SemaphoreType.DMA,
    ],
)
def _colgather(xt_hbm, tt_hbm, out_hbm, idx_v, gb0, gb1, colA,
               sg0, sg1, sw0, sw1, scol):
    cid = lax.axis_index("c")
    sid = lax.axis_index("s")
    p0 = pl.multiple_of(sid * PPT, PPT)
    pltpu.sync_copy(xt_hbm.at[pl.ds(p0, PPT)], idx_v)

    gbufs = (gb0, gb1)
    gsems = (sg0, sg1)
    wsems = (sw0, sw1)
    jbase = cid * CPS

    def load_col(jj):
        pltpu.async_copy(tt_hbm.at[jbase + jj], colA, scol)

    def wait_col(jj):
        pltpu.make_async_copy(tt_hbm.at[jbase + jj], colA, scol).wait()

    @pl.when(sid == 0)
    def _():
        load_col(0)
        wait_col(0)

    @pl.loop(0, CPS)
    def _cols(jj):
        plsc.subcore_barrier()
        j = jbase + jj

        def g_start(k, b):
            pltpu.async_copy(
                colA.at[idx_v.at[pl.ds(pl.multiple_of(k * CB, CB), CB)]],
                gbufs[b], gsems[b],
            )

        def g_wait(k, b):
            pltpu.make_async_copy(
                colA.at[idx_v.at[pl.ds(pl.multiple_of(k * CB, CB), CB)]],
                gbufs[b], gsems[b],
            ).wait()

        def w_dst(k):
            p = p0 + k * CB
            r = (p >> 12) * D + j
            b1 = pl.multiple_of(p & (B1 - 1), CB)
            return out_hbm.at[r, pl.ds(b1, CB)]

        def w_start(k, b):
            pltpu.async_copy(gbufs[b], w_dst(k), wsems[b])

        def w_wait(k, b):
            pltpu.make_async_copy(gbufs[b], w_dst(k), wsems[b]).wait()

        g_start(0, 0)
        g_start(1, 1)

        @pl.loop(0, NCH - 1, step=2)
        def _chunks(k):
            g_wait(k, 0)
            w_start(k, 0)
            g_wait(k + 1, 1)
            w_start(k + 1, 1)
            w_wait(k, 0)
            g_start(k + 2, 0)

            @pl.when(k + 3 < NCH)
            def _():
                w_wait(k + 1, 1)
                g_start(k + 3, 1)

        g_wait(NCH - 1, 0)
        plsc.subcore_barrier()

        @pl.when(sid == 0)
        def _():
            @pl.when(jj + 1 < CPS)
            def _():
                load_col(jj + 1)

        w_start(NCH - 1, 0)
        w_wait(NCH - 2, 1)
        w_wait(NCH - 1, 0)

        @pl.when(sid == 0)
        def _():
            @pl.when(jj + 1 < CPS)
            def _():
                wait_col(jj + 1)


def kernel(x, table):
    xt = x.T.reshape(-1)
    tt = table.T
    out2 = _colgather(xt, tt)
    out3 = out2.reshape(B2, D, B1)
    return jnp.transpose(out3, (2, 0, 1))


# speedup vs baseline: 8.6489x; 1.1365x over previous
import functools

import jax
import jax.numpy as jnp
from jax import lax
from jax.experimental import pallas as pl
from jax.experimental.pallas import tpu as pltpu
from jax.experimental.pallas import tpu_sc as plsc

NC, NS = 2, 16
V = 1000000
D = 64
B1, B2 = 4096, 200
B = B1 * B2
CPS = D // NC
PPT = B // NS
CB = 2048
NCH = PPT // CB

_MESH = plsc.VectorSubcoreMesh(
    core_axis_name="c", subcore_axis_name="s", num_cores=NC, num_subcores=NS
)


@functools.partial(
    pl.kernel,
    out_type=jax.ShapeDtypeStruct((B2 * D, B1), jnp.float32),
    mesh=_MESH,
    compiler_params=pltpu.CompilerParams(use_tc_tiling_on_sc=True),
    scratch_types=[
        pltpu.VMEM((PPT,), jnp.int32),
        pltpu.VMEM((CB,), jnp.float32),
        pltpu.VMEM((CB,), jnp.float32),
        pltpu.VMEM((CB,), jnp.float32),
        pltpu.VMEM((CB,), jnp.float32),
        pltpu.VMEM_SHARED((V,), jnp.float32),
        pltpu.SemaphoreType.DMA,
        pltpu.SemaphoreType.DMA,
        pltpu.SemaphoreType.DMA,
        pltpu.SemaphoreType.DMA,
        pltpu.SemaphoreType.DMA,
        pltpu.SemaphoreType.DMA,
        pltpu.SemaphoreType.DMA,
        pltpu.SemaphoreType.DMA,
        pltpu.SemaphoreType.DMA,
    ],
)
def _colgather(xt_hbm, tt_hbm, out_hbm, idx_v, gb0, gb1, gb2, gb3, colA,
               sg0, sg1, sg2, sg3, sw0, sw1, sw2, sw3, scol):
    cid = lax.axis_index("c")
    sid = lax.axis_index("s")
    p0 = pl.multiple_of(sid * PPT, PPT)
    pltpu.sync_copy(xt_hbm.at[pl.ds(p0, PPT)], idx_v)

    gbufs = (gb0, gb1, gb2, gb3)
    gsems = (sg0, sg1, sg2, sg3)
    wsems = (sw0, sw1, sw2, sw3)
    jbase = cid * CPS

    def load_col(jj):
        pltpu.async_copy(tt_hbm.at[jbase + jj], colA, scol)

    def wait_col(jj):
        pltpu.make_async_copy(tt_hbm.at[jbase + jj], colA, scol).wait()

    @pl.when(sid == 0)
    def _():
        load_col(0)
        wait_col(0)

    @pl.loop(0, CPS)
    def _cols(jj):
        plsc.subcore_barrier()
        j = jbase + jj

        def g_start(k, b):
            pltpu.async_copy(
                colA.at[idx_v.at[pl.ds(pl.multiple_of(k * CB, CB), CB)]],
                gbufs[b], gsems[b],
            )

        def g_wait(k, b):
            pltpu.make_async_copy(
                colA.at[idx_v.at[pl.ds(pl.multiple_of(k * CB, CB), CB)]],
                gbufs[b], gsems[b],
            ).wait()

        def w_dst(k):
            p = p0 + k * CB
            r = (p >> 12) * D + j
            b1 = pl.multiple_of(p & (B1 - 1), CB)
            return out_hbm.at[r, pl.ds(b1, CB)]

        def w_start(k, b):
            pltpu.async_copy(gbufs[b], w_dst(k), wsems[b])

        def w_wait(k, b):
            pltpu.make_async_copy(gbufs[b], w_dst(k), wsems[b]).wait()

        for b in range(4):
            g_start(b, b)

        @pl.loop(0, NCH - 1, step=4)
        def _chunks(k):
            for q in range(4):
                g_wait(k + q, q)
                w_start(k + q, q)
            for q in range(4):
                @pl.when(k + 4 + q < NCH)
                def _(q=q):
                    w_wait(k + q, q)
                    g_start(k + 4 + q, q)

        g_wait(NCH - 1, 0)
        plsc.subcore_barrier()

        @pl.when(sid == 0)
        def _():
            @pl.when(jj + 1 < CPS)
            def _():
                load_col(jj + 1)

        w_start(NCH - 1, 0)
        for q in range(1, 4):
            w_wait(NCH - 5 + q, q)
        w_wait(NCH - 1, 0)

        @pl.when(sid == 0)
        def _():
            @pl.when(jj + 1 < CPS)
            def _():
                wait_col(jj + 1)


def kernel(x, table):
    xt = x.T.reshape(-1)
    tt = table.T
    out2 = _colgather(xt, tt)
    out3 = out2.reshape(B2, D, B1)
    return jnp.transpose(out3, (2, 0, 1))
